# baseline (speedup 1.0000x reference)
import functools

import jax
import jax.numpy as jnp
from jax.experimental import pallas as pl
from jax.experimental.pallas import tpu as pltpu


def _head_kernel(pooled_ref, f1_ref, bf1_ref, f2_ref, bf2_ref, out_ref):
    hfc = jnp.maximum(pooled_ref[...], 0.0)
    z = jnp.dot(hfc, f1_ref[...], preferred_element_type=jnp.float32)
    z = jax.nn.sigmoid(z + bf1_ref[...])
    out_ref[...] = jnp.dot(z, f2_ref[...], preferred_element_type=jnp.float32) + bf2_ref[...]


def _fc_head(pooled, F1, bf1, F2, bf2):
    G = pooled.shape[0]
    return pl.pallas_call(
        _head_kernel,
        out_shape=jax.ShapeDtypeStruct((G, F2.shape[1]), jnp.float32),
    )(pooled, F1, bf1[None, :], F2, bf2[None, :])


def kernel(x, edge_index, edge_attr, batch, W, We, a_src, a_dst, a_edge, b_enc,
           Gg1, bg1, Gg2, bg2, Gn1, bn1, Gn2, bn2, F1, bf1, F2, bf2):
    N = x.shape[0]
    G = 64
    src = edge_index[0]
    dst = edge_index[1]

    h = x @ W
    s_sd = h @ jnp.stack([a_src, a_dst], axis=1)
    t = edge_attr @ (We @ a_edge)

    logits = s_sd[src, 0] + s_sd[dst, 1] + t
    logits = jax.nn.leaky_relu(logits, 0.2)
    M = jnp.max(logits)
    e_val = jnp.exp(logits - M)

    ssum = jax.ops.segment_sum(e_val, dst, num_segments=N)
    acc = jax.ops.segment_sum(e_val[:, None] * jnp.take(h, src, axis=0),
                              dst, num_segments=N)
    eacc = jax.ops.segment_sum(e_val[:, None] * edge_attr,
                               dst, num_segments=N)

    x_enc = (acc + eacc @ We) / (ssum[:, None] + 1e-16) + b_enc
    x_enc = jax.nn.elu(x_enc)

    gate = jax.nn.relu(x_enc @ Gg1 + bg1) @ Gg2 + bg2
    val = jax.nn.relu(x_enc @ Gn1 + bn1) @ Gn2 + bn2

    g = gate[:, 0]
    gm = jax.ops.segment_max(g, batch, num_segments=G)
    gm = jnp.where(jnp.isfinite(gm), gm, 0.0)
    ge = jnp.exp(g - gm[batch])
    gs = jax.ops.segment_sum(ge, batch, num_segments=G)
    a = ge / (gs[batch] + 1e-16)
    pooled = jax.ops.segment_sum(a[:, None] * val, batch, num_segments=G)

    return _fc_head(pooled, F1, bf1, F2, bf2)


# speedup vs baseline: 3.7748x; 3.7748x over previous
import dataclasses
import functools

import jax
import jax.numpy as jnp
from jax import lax
from jax.experimental import pallas as pl
from jax.experimental.pallas import tpu as pltpu
from jax.experimental.pallas import tpu_sc as plsc

N = 10000
E = 320000
EMB = 1024
DE = 16
NW = 32
EPT = E // NW
KB = 64
NB = 160
EPT_P = NB * KB
NP = N + 112
DUMMY = N + 8
ROWS_PT = NP // 16
NCHUNK = 8


def _sc_compiler_params():
    cp = pltpu.CompilerParams()
    if "needs_layout_passes" in pltpu.CompilerParams.__dataclass_fields__:
        cp = dataclasses.replace(cp, needs_layout_passes=False)
    return cp


def _a_body(x_ref, w_ref, asd_ref, h8_ref, ssd_ref):
    c = pl.program_id(1)
    hc = jnp.dot(x_ref[...], w_ref[0], preferred_element_type=jnp.float32)
    h8_ref[...] = hc[None]

    @pl.when(c == 0)
    def _():
        ssd_ref[...] = jnp.zeros_like(ssd_ref)

    ssd_ref[...] += jnp.dot(hc, asd_ref[...], preferred_element_type=jnp.float32)


def _kernel_a(x, W, a_sd):
    bn = 1000
    W3 = W.reshape(128, NCHUNK, 128).transpose(1, 0, 2)
    return pl.pallas_call(
        _a_body,
        grid=(N // bn, NCHUNK),
        in_specs=[
            pl.BlockSpec((bn, 128), lambda i, c: (i, 0)),
            pl.BlockSpec((1, 128, 128), lambda i, c: (c, 0, 0)),
            pl.BlockSpec((128, 2), lambda i, c: (c, 0)),
        ],
        out_specs=[
            pl.BlockSpec((1, bn, 128), lambda i, c: (c, i, 0)),
            pl.BlockSpec((bn, 2), lambda i, c: (i, 0)),
        ],
        out_shape=[
            jax.ShapeDtypeStruct((NCHUNK, N, 128), jnp.float32),
            jax.ShapeDtypeStruct((N, 2), jnp.float32),
        ],
    )(x, W3, a_sd)


def _b_body(ssrc_hbm, sdst_hbm, pck_hbm, lsd_hbm,
            ssrc_v, sdst_v, pck_v, l_v):
    wid = lax.axis_index("s") * 2 + lax.axis_index("c")
    pltpu.sync_copy(ssrc_hbm, ssrc_v)
    pltpu.sync_copy(sdst_hbm, sdst_v)
    pltpu.sync_copy(pck_hbm.at[wid], pck_v)

    @pl.loop(0, EPT_P, step=16)
    def _(i):
        p = pck_v[pl.ds(i, 16)]
        si = lax.bitwise_and(p, 0x3FFF)
        di = lax.bitwise_and(lax.shift_right_logical(p, 14), 0x3FFF)
        a = plsc.load_gather(ssrc_v, [si])
        b = plsc.load_gather(sdst_v, [di])
        l_v[pl.ds(i, 16)] = a + b

    pltpu.sync_copy(l_v, lsd_hbm.at[wid])


def _kernel_b(s_src, s_dst, pck):
    mesh = plsc.VectorSubcoreMesh(core_axis_name="c", subcore_axis_name="s")
    f = pl.kernel(
        _b_body,
        out_type=jax.ShapeDtypeStruct((NW, EPT_P), jnp.float32),
        mesh=mesh,
        scratch_types=[
            pltpu.VMEM((N,), jnp.float32),
            pltpu.VMEM((N,), jnp.float32),
            pltpu.VMEM((EPT_P,), jnp.int32),
            pltpu.VMEM((EPT_P,), jnp.float32),
        ],
        compiler_params=_sc_compiler_params(),
    )
    return f(s_src, s_dst, pck)


def _a2_body(eap_ref, we_ref, ae_ref, t8_ref):
    wea = jnp.dot(we_ref[...], ae_ref[...], preferred_element_type=jnp.float32)
    wea_t = jnp.broadcast_to(wea.reshape(1, DE), (8, DE)).reshape(128, 1)
    rowi = lax.broadcasted_iota(jnp.int32, (128, 8), 0)
    coli = lax.broadcasted_iota(jnp.int32, (128, 8), 1)
    wea8 = jnp.where(coli == rowi // DE, wea_t, 0.0)
    t8_ref[...] = jnp.dot(eap_ref[...], wea8, preferred_element_type=jnp.float32)


def _kernel_a2(eap2, We, a_edge):
    nr = NW * EPT_P // 8
    bn = nr // 10
    return pl.pallas_call(
        _a2_body,
        grid=(10,),
        in_specs=[
            pl.BlockSpec((bn, 128), lambda i: (i, 0)),
            pl.BlockSpec((DE, EMB), lambda i: (0, 0)),
            pl.BlockSpec((EMB, 1), lambda i: (0, 0)),
        ],
        out_specs=pl.BlockSpec((bn, 8), lambda i: (i, 0)),
        out_shape=jax.ShapeDtypeStruct((nr, 8), jnp.float32),
    )(eap2, We, a_edge)


def _c_body(lsd_ref, t_ref, ev_ref):
    l = lsd_ref[...] + t_ref[...]
    l = jnp.where(l >= 0.0, l, 0.2 * l)
    m = jnp.max(l)
    ev_ref[...] = jnp.exp(l - m)


def _kernel_c(lsd, t2d):
    return pl.pallas_call(
        _c_body,
        out_shape=jax.ShapeDtypeStruct((NW, EPT_P), jnp.float32),
    )(lsd, t2d)


def _zero_rows(row_v):
    nq = row_v.shape[1] // 16

    @pl.loop(0, row_v.shape[0])
    def _(rz):
        for q in range(nq):
            row_v[rz, pl.ds(16 * q, 16)] = jnp.zeros((16,), jnp.float32)


def _clear_accS(sid, row_v, accS):
    rows0 = sid * ROWS_PT
    _zero_rows(row_v)
    for tb in range(10):
        hgt = KB if tb < 9 else ROWS_PT - 9 * KB
        pltpu.sync_copy(row_v.at[pl.ds(0, hgt)],
                        accS.at[pl.ds(rows0 + KB * tb, hgt)])
    plsc.subcore_barrier()


def _unpack_idx(pck_v, j, srcb, dstb):
    slot = lax.rem(lax.div(j, 8), 2)
    jj = lax.rem(j, 8)
    for q in range(KB // 16):
        p = pck_v[slot, jj, pl.ds(16 * q, 16)]
        srcb[pl.ds(16 * q, 16)] = lax.bitwise_and(p, 0x3FFF)
        dstb[pl.ds(16 * q, 16)] = lax.bitwise_and(
            lax.shift_right_logical(p, 14), 0x3FFF)


def _load_pck_chunk(pck_hbm, r, pck_v, cidx):
    pltpu.sync_copy(pck_hbm.at[r, pl.ds(cidx * 8, 8)],
                    pck_v.at[lax.rem(cidx, 2)])


def _mult_rows(row, ev_v, j):
    @pl.loop(0, KB)
    def _(k):
        w = plsc.load_gather(ev_v, [jnp.full((16,), j, jnp.int32),
                                    jnp.full((16,), k, jnp.int32)])
        for q in range(8):
            row[k, pl.ds(16 * q, 16)] = row[k, pl.ds(16 * q, 16)] * w


def _e_round(c, sid, h8_hbm, pck_hbm, evs_hbm, acc_hbm, accS, ev_v, pck_v,
             row0, row1, srcb0, dstb0, srcb1, dstb1, g0, g1, s0, s1):
    _clear_accS(sid, row0, accS)
    hc = h8_hbm.at[c]

    def g_start(row, srcb, sem):
        pltpu.async_copy(hc.at[srcb], row, sem)

    def g_wait(row, srcb, sem):
        pltpu.make_async_copy(hc.at[srcb], row, sem).wait()

    def s_start(row, dstb, sem):
        pltpu.async_copy(row, accS.at[dstb], sem, add=True)

    def s_wait(row, dstb, sem):
        pltpu.make_async_copy(row, accS.at[dstb], sem).wait()

    for rr in range(2):
        r = sid * 2 + rr
        pltpu.sync_copy(evs_hbm.at[r], ev_v)
        _load_pck_chunk(pck_hbm, r, pck_v, 0)
        _load_pck_chunk(pck_hbm, r, pck_v, 1)

        _unpack_idx(pck_v, 0, srcb0, dstb0)
        g_start(row0, srcb0, g0)

        @pl.loop(0, NB, step=2)
        def _(j):
            @pl.when(j > 0)
            def _():
                s_wait(row1, dstb1, s1)

            _unpack_idx(pck_v, j + 1, srcb1, dstb1)
            g_start(row1, srcb1, g1)
            g_wait(row0, srcb0, g0)
            _mult_rows(row0, ev_v, j)
            s_start(row0, dstb0, s0)
            g_wait(row1, srcb1, g1)
            _mult_rows(row1, ev_v, j + 1)
            s_start(row1, dstb1, s1)
            s_wait(row0, dstb0, s0)

            @pl.when((lax.rem(j + 2, 8) == 0) & (j >= 14) & (j + 2 < NB))
            def _():
                _load_pck_chunk(pck_hbm, r, pck_v, lax.div(j + 2, 8))

            @pl.when(j + 2 < NB)
            def _():
                _unpack_idx(pck_v, j + 2, srcb0, dstb0)
                g_start(row0, srcb0, g0)

        s_wait(row1, dstb1, s1)

    plsc.subcore_barrier()
    rows0 = sid * ROWS_PT
    for tb in range(10):
        hgt = KB if tb < 9 else ROWS_PT - 9 * KB
        pltpu.sync_copy(accS.at[pl.ds(rows0 + KB * tb, hgt)],
                        acc_hbm.at[c, pl.ds(rows0 + KB * tb, hgt)])
    plsc.subcore_barrier()


def _e_extras(sid, eap_hbm, pck_hbm, evs_hbm, ex_hbm, accS, ev_v, pck_v,
              row_v, eb_v, srcb, dstb):
    _clear_accS(sid, row_v, accS)

    lane0 = jnp.where(lax.iota(jnp.int32, 16) == 0, 1.0, 0.0)
    for rr in range(2):
        r = sid * 2 + rr
        pltpu.sync_copy(evs_hbm.at[r], ev_v)

        @pl.loop(0, NB)
        def _(j):
            @pl.when(lax.rem(j, 8) == 0)
            def _():
                _load_pck_chunk(pck_hbm, r, pck_v, lax.div(j, 8))

            pltpu.sync_copy(
                eap_hbm.at[pl.ds(r * (EPT_P // 8) + j * (KB // 8), KB // 8)],
                eb_v)
            _unpack_idx(pck_v, j, srcb, dstb)

            @pl.loop(0, KB)
            def _(k):
                w = plsc.load_gather(ev_v, [jnp.full((16,), j, jnp.int32),
                                            jnp.full((16,), k, jnp.int32)])
                ea16 = eb_v[lax.div(k, 8), pl.ds(DE * lax.rem(k, 8), DE)]
                row_v[k, pl.ds(0, 16)] = ea16 * w
                row_v[k, pl.ds(16, 16)] = w * lane0

            pltpu.sync_copy(row_v, accS.at[dstb], add=True)

    plsc.subcore_barrier()
    rows0 = sid * ROWS_PT
    for tb in range(10):
        hgt = KB if tb < 9 else ROWS_PT - 9 * KB
        pltpu.sync_copy(accS.at[pl.ds(rows0 + KB * tb, hgt)],
                        ex_hbm.at[pl.ds(rows0 + KB * tb, hgt)])
    plsc.subcore_barrier()


def _e_body(h8_hbm, pck_hbm, evs_hbm, eap_hbm, acc_hbm, ex_hbm,
            accS, pck_v, ev_v, row0, row1, eb_v, srcb0, dstb0, srcb1, dstb1,
            g0, g1, s0, s1):
    cid = lax.axis_index("c")
    sid = lax.axis_index("s")

    for c in range(NCHUNK):
        @pl.when(cid == c // (NCHUNK // 2))
        def _():
            _e_round(c, sid, h8_hbm, pck_hbm, evs_hbm, acc_hbm, accS,
                     ev_v, pck_v,
                     row0, row1, srcb0, dstb0, srcb1, dstb1, g0, g1, s0, s1)

    @pl.when(cid == 0)
    def _():
        _e_extras(sid, eap_hbm, pck_hbm, evs_hbm, ex_hbm, accS, ev_v, pck_v,
                  row_v=row0, eb_v=eb_v, srcb=srcb0, dstb=dstb0)


def _kernel_e(h8, pck3, evs3, eap3):
    mesh = plsc.VectorSubcoreMesh(core_axis_name="c", subcore_axis_name="s")
    f = pl.kernel(
        _e_body,
        out_type=[
            jax.ShapeDtypeStruct((NCHUNK, NP, 128), jnp.float32),
            jax.ShapeDtypeStruct((NP, 128), jnp.float32),
        ],
        mesh=mesh,
        scratch_types=[
            pltpu.VMEM_SHARED((NP, 128), jnp.float32),
            pltpu.VMEM((2, 8, KB), jnp.int32),
            pltpu.VMEM((NB, KB), jnp.float32),
            pltpu.VMEM((KB, 128), jnp.float32),
            pltpu.VMEM((KB, 128), jnp.float32),
            pltpu.VMEM((KB // 8, 128), jnp.float32),
            pltpu.VMEM((KB,), jnp.int32),
            pltpu.VMEM((KB,), jnp.int32),
            pltpu.VMEM((KB,), jnp.int32),
            pltpu.VMEM((KB,), jnp.int32),
            pltpu.SemaphoreType.DMA,
            pltpu.SemaphoreType.DMA,
            pltpu.SemaphoreType.DMA,
            pltpu.SemaphoreType.DMA,
        ],
        compiler_params=_sc_compiler_params(),
    )
    return f(h8, pck3, evs3, eap3)


def _f1_body(acc_ref, ex_ref, we_ref, benc_ref,
             gg1_ref, bg1_ref, gg2_ref, bg2_ref,
             gn1_ref, bn1_ref, gn2_ref, bn2_ref,
             gate_ref, val_ref):
    ex = ex_ref[...]
    num = acc_ref[...] + jnp.dot(ex[:, :DE], we_ref[...],
                                 preferred_element_type=jnp.float32)
    den = ex[:, DE:DE + 1] + 1e-16
    xe = num / den + benc_ref[...]
    xe = jnp.where(xe > 0.0, xe, jnp.exp(jnp.minimum(xe, 0.0)) - 1.0)
    hg = jnp.maximum(jnp.dot(xe, gg1_ref[...], preferred_element_type=jnp.float32)
                     + bg1_ref[...], 0.0)
    gate_ref[...] = jnp.dot(hg, gg2_ref[...], preferred_element_type=jnp.float32) + bg2_ref[...]
    hv = jnp.maximum(jnp.dot(xe, gn1_ref[...], preferred_element_type=jnp.float32)
                     + bn1_ref[...], 0.0)
    val_ref[...] = jnp.dot(hv, gn2_ref[...], preferred_element_type=jnp.float32) + bn2_ref[...]


def _kernel_f1(acc, extras, We, b_enc, Gg1, bg1, Gg2, bg2, Gn1, bn1, Gn2, bn2):
    bn = 1000
    full = lambda a, b: pl.BlockSpec((a, b), lambda i: (0, 0))
    return pl.pallas_call(
        _f1_body,
        grid=(N // bn,),
        in_specs=[
            pl.BlockSpec((bn, EMB), lambda i: (i, 0)),
            pl.BlockSpec((bn, 32), lambda i: (i, 0)),
            full(DE, EMB), full(1, EMB),
            full(EMB, 512), full(1, 512), full(512, 1), full(1, 1),
            full(EMB, 512), full(1, 512), full(512, EMB), full(1, EMB),
        ],
        out_specs=[
            pl.BlockSpec((bn, 1), lambda i: (i, 0)),
            pl.BlockSpec((bn, EMB), lambda i: (i, 0)),
        ],
        out_shape=[
            jax.ShapeDtypeStruct((N, 1), jnp.float32),
            jax.ShapeDtypeStruct((N, EMB), jnp.float32),
        ],
    )(acc, extras, We, b_enc, Gg1, bg1, Gg2, bg2, Gn1, bn1, Gn2, bn2)


def _f2_body(gate_ref, batch_ref, val_ref, f1_ref, bf1_ref, f2_ref, bf2_ref,
             out_ref):
    g = gate_ref[...]
    m = jnp.max(g)
    ge = jnp.exp(g - m)
    gid = lax.broadcasted_iota(jnp.int32, (64, 1), 0)
    wmat = (batch_ref[...] == gid).astype(jnp.float32) * ge
    sseg = jnp.sum(wmat, axis=1, keepdims=True)
    pooled = jnp.dot(wmat, val_ref[...],
                     preferred_element_type=jnp.float32) / (sseg + 1e-16)
    hfc = jnp.maximum(pooled, 0.0)
    z = jax.nn.sigmoid(jnp.dot(hfc, f1_ref[...],
                               preferred_element_type=jnp.float32) + bf1_ref[...])
    out_ref[...] = jnp.dot(z, f2_ref[...],
                           preferred_element_type=jnp.float32) + bf2_ref[...]


def _kernel_f2(gate_row, batch_row, val, F1, bf1, F2, bf2):
    return pl.pallas_call(
        _f2_body,
        out_shape=jax.ShapeDtypeStruct((64, 3), jnp.float32),
    )(gate_row, batch_row, val, F1, bf1, F2, bf2)


def kernel(x, edge_index, edge_attr, batch, W, We, a_src, a_dst, a_edge, b_enc,
           Gg1, bg1, Gg2, bg2, Gn1, bn1, Gn2, bn2, F1, bf1, F2, bf2):
    src = edge_index[0]
    dst = edge_index[1]
    pad = EPT_P - EPT
    srcp = jnp.pad(src.reshape(NW, EPT), ((0, 0), (0, pad)))
    dstp = jnp.pad(dst.reshape(NW, EPT), ((0, 0), (0, pad)),
                   constant_values=DUMMY)
    pck = srcp | (dstp << 14)
    eap2 = jnp.pad(edge_attr.reshape(NW, EPT, DE),
                   ((0, 0), (0, pad), (0, 0))).reshape(NW * EPT_P * DE // 128,
                                                       128)
    a_sd = jnp.stack([a_src, a_dst], axis=1)

    h8, s_sd = _kernel_a(x, W, a_sd)
    lsd = _kernel_b(s_sd[:, 0], s_sd[:, 1], pck)
    t8 = _kernel_a2(eap2, We, a_edge.reshape(EMB, 1))
    ev = _kernel_c(lsd, t8.reshape(NW, EPT_P))
    acc3, extras = _kernel_e(h8,
                             pck.reshape(NW, NB, KB),
                             ev.reshape(NW, NB, KB),
                             eap2)
    acc = acc3.transpose(1, 0, 2).reshape(NP, EMB)
    gate, val = _kernel_f1(acc[:N], extras[:N, :32], We, b_enc.reshape(1, EMB),
                           Gg1, bg1.reshape(1, 512), Gg2, bg2.reshape(1, 1),
                           Gn1, bn1.reshape(1, 512), Gn2, bn2.reshape(1, EMB))
    out = _kernel_f2(gate.reshape(1, N), batch.reshape(1, N), val,
                     F1, bf1.reshape(1, 512), F2, bf2.reshape(1, 3))
    return out


# speedup vs baseline: 3.9809x; 1.0546x over previous
import dataclasses
import functools

import numpy as np

import jax
import jax.numpy as jnp
from jax import lax
from jax.experimental import pallas as pl
from jax.experimental.pallas import tpu as pltpu
from jax.experimental.pallas import tpu_sc as plsc

N = 10000
E = 320000
EMB = 1024
DE = 16
NW = 32
EPT = E // NW
KB = 64
NB = 160
EPT_P = NB * KB
NP = N + 112
DUMMY = N + 8
ROWS_PT = NP // 16
NCHUNK = 8

_PG = np.arange(32).reshape(2, 16).T.reshape(-1)
_PERM = (np.arange(0, EMB, 32)[:, None] + _PG[None, :]).reshape(-1)
_IPERM = np.argsort(_PERM)


def _sc_compiler_params():
    cp = pltpu.CompilerParams()
    if "needs_layout_passes" in pltpu.CompilerParams.__dataclass_fields__:
        cp = dataclasses.replace(cp, needs_layout_passes=False)
    return cp


def _a_body(x_ref, w_ref, asd_ref, h8_ref, ssd_ref):
    c = pl.program_id(1)
    hc = jnp.dot(x_ref[...], w_ref[0], preferred_element_type=jnp.float32)
    h8_ref[...] = hc[None]

    @pl.when(c == 0)
    def _():
        ssd_ref[...] = jnp.zeros_like(ssd_ref)

    ssd_ref[...] += jnp.dot(hc, asd_ref[...], preferred_element_type=jnp.float32)


def _kernel_a(x, W, a_sd):
    bn = 1000
    W3 = W.reshape(128, NCHUNK, 128).transpose(1, 0, 2)
    return pl.pallas_call(
        _a_body,
        grid=(N // bn, NCHUNK),
        in_specs=[
            pl.BlockSpec((bn, 128), lambda i, c: (i, 0)),
            pl.BlockSpec((1, 128, 128), lambda i, c: (c, 0, 0)),
            pl.BlockSpec((128, 2), lambda i, c: (c, 0)),
        ],
        out_specs=[
            pl.BlockSpec((1, bn, 128), lambda i, c: (c, i, 0)),
            pl.BlockSpec((bn, 2), lambda i, c: (i, 0)),
        ],
        out_shape=[
            jax.ShapeDtypeStruct((NCHUNK, N, 128), jnp.float32),
            jax.ShapeDtypeStruct((N, 2), jnp.float32),
        ],
    )(x, W3, a_sd)


def _b_body(ssrc_hbm, sdst_hbm, pck_hbm, lsd_hbm,
            ssrc_v, sdst_v, pck_v, l_v):
    wid = lax.axis_index("s") * 2 + lax.axis_index("c")
    pltpu.sync_copy(ssrc_hbm, ssrc_v)
    pltpu.sync_copy(sdst_hbm, sdst_v)
    pltpu.sync_copy(pck_hbm.at[wid], pck_v)

    @pl.loop(0, EPT_P, step=16)
    def _(i):
        p = pck_v[pl.ds(i, 16)]
        si = lax.bitwise_and(p, 0x3FFF)
        di = lax.bitwise_and(lax.shift_right_logical(p, 14), 0x3FFF)
        a = plsc.load_gather(ssrc_v, [si])
        b = plsc.load_gather(sdst_v, [di])
        l_v[pl.ds(i, 16)] = a + b

    pltpu.sync_copy(l_v, lsd_hbm.at[wid])


def _kernel_b(s_src, s_dst, pck):
    mesh = plsc.VectorSubcoreMesh(core_axis_name="c", subcore_axis_name="s")
    f = pl.kernel(
        _b_body,
        out_type=jax.ShapeDtypeStruct((NW, EPT_P), jnp.float32),
        mesh=mesh,
        scratch_types=[
            pltpu.VMEM((N,), jnp.float32),
            pltpu.VMEM((N,), jnp.float32),
            pltpu.VMEM((EPT_P,), jnp.int32),
            pltpu.VMEM((EPT_P,), jnp.float32),
        ],
        compiler_params=_sc_compiler_params(),
    )
    return f(s_src, s_dst, pck)


def _a2_body(eap_ref, we_ref, ae_ref, t8_ref):
    wea = jnp.dot(we_ref[...], ae_ref[...], preferred_element_type=jnp.float32)
    wea_t = jnp.broadcast_to(wea.reshape(1, DE), (8, DE)).reshape(128, 1)
    rowi = lax.broadcasted_iota(jnp.int32, (128, 8), 0)
    coli = lax.broadcasted_iota(jnp.int32, (128, 8), 1)
    wea8 = jnp.where(coli == rowi // DE, wea_t, 0.0)
    t8_ref[...] = jnp.dot(eap_ref[...], wea8, preferred_element_type=jnp.float32)


def _kernel_a2(eap2, We, a_edge):
    nr = NW * EPT_P // 8
    bn = nr // 10
    return pl.pallas_call(
        _a2_body,
        grid=(10,),
        in_specs=[
            pl.BlockSpec((bn, 128), lambda i: (i, 0)),
            pl.BlockSpec((DE, EMB), lambda i: (0, 0)),
            pl.BlockSpec((EMB, 1), lambda i: (0, 0)),
        ],
        out_specs=pl.BlockSpec((bn, 8), lambda i: (i, 0)),
        out_shape=jax.ShapeDtypeStruct((nr, 8), jnp.float32),
    )(eap2, We, a_edge)


def _c_body(lsd_ref, t_ref, ev_ref):
    l = lsd_ref[...] + t_ref[...]
    l = jnp.where(l >= 0.0, l, 0.2 * l)
    m = jnp.max(l)
    ev_ref[...] = jnp.exp(l - m)


def _kernel_c(lsd, t2d):
    return pl.pallas_call(
        _c_body,
        out_shape=jax.ShapeDtypeStruct((NW, EPT_P), jnp.float32),
    )(lsd, t2d)


def _zero_rows(row_v):
    nq = row_v.shape[1] // 16

    @pl.loop(0, row_v.shape[0])
    def _(rz):
        for q in range(nq):
            row_v[rz, pl.ds(16 * q, 16)] = jnp.zeros((16,), jnp.float32)


def _clear_accS(sid, row_v, accS):
    rows0 = sid * ROWS_PT
    _zero_rows(row_v)
    for tb in range(10):
        hgt = 64 if tb < 9 else ROWS_PT - 9 * 64
        pltpu.sync_copy(row_v.at[pl.ds(0, hgt)],
                        accS.at[pl.ds(rows0 + 64 * tb, hgt)])
    plsc.subcore_barrier()


def _unpack_idx(pck_v, j, srcb, dstb):
    slot = lax.rem(lax.div(j, 8), 2)
    jj = lax.rem(j, 8)
    for q in range(KB // 16):
        p = pck_v[slot, jj, pl.ds(16 * q, 16)]
        srcb[pl.ds(16 * q, 16)] = lax.bitwise_and(p, 0x3FFF)
        dstb[pl.ds(16 * q, 16)] = lax.bitwise_and(
            lax.shift_right_logical(p, 14), 0x3FFF)


def _load_pck_chunk(pck_hbm, r, pck_v, cidx):
    pltpu.sync_copy(pck_hbm.at[r, pl.ds(cidx * 8, 8)],
                    pck_v.at[lax.rem(cidx, 2)])


def _mult_rows(row, ev_v, j):
    @pl.loop(0, KB)
    def _(k):
        w = plsc.load_gather(ev_v, [jnp.full((16,), j, jnp.int32),
                                    jnp.full((16,), k, jnp.int32)])
        for q in range(8):
            row[k, pl.ds(16 * q, 16)] = row[k, pl.ds(16 * q, 16)] * w


def _e_round(c, sid, h8_hbm, pck_hbm, evs_hbm, acc_hbm, accS, ev_v, pck_v,
             row0, row1, srcb0, dstb0, srcb1, dstb1, g0, g1, s0, s1):
    _clear_accS(sid, row0, accS)
    hc = h8_hbm.at[c]

    def g_start(row, srcb, sem):
        pltpu.async_copy(hc.at[srcb], row, sem)

    def g_wait(row, srcb, sem):
        pltpu.make_async_copy(hc.at[srcb], row, sem).wait()

    def s_start(row, dstb, sem):
        pltpu.async_copy(row, accS.at[dstb], sem, add=True)

    def s_wait(row, dstb, sem):
        pltpu.make_async_copy(row, accS.at[dstb], sem).wait()

    for rr in range(2):
        r = sid * 2 + rr
        pltpu.sync_copy(evs_hbm.at[r], ev_v)
        _load_pck_chunk(pck_hbm, r, pck_v, 0)
        _load_pck_chunk(pck_hbm, r, pck_v, 1)

        _unpack_idx(pck_v, 0, srcb0, dstb0)
        g_start(row0, srcb0, g0)

        @pl.loop(0, NB, step=2)
        def _(j):
            @pl.when(j > 0)
            def _():
                s_wait(row1, dstb1, s1)

            _unpack_idx(pck_v, j + 1, srcb1, dstb1)
            g_start(row1, srcb1, g1)
            g_wait(row0, srcb0, g0)
            _mult_rows(row0, ev_v, j)
            s_start(row0, dstb0, s0)
            g_wait(row1, srcb1, g1)
            _mult_rows(row1, ev_v, j + 1)
            s_start(row1, dstb1, s1)

            @pl.when((lax.rem(j + 2, 8) == 0) & (j >= 14) & (j + 2 < NB))
            def _():
                _load_pck_chunk(pck_hbm, r, pck_v, lax.div(j + 2, 8))

            s_wait(row0, dstb0, s0)

            @pl.when(j + 2 < NB)
            def _():
                _unpack_idx(pck_v, j + 2, srcb0, dstb0)
                g_start(row0, srcb0, g0)

        s_wait(row1, dstb1, s1)

    plsc.subcore_barrier()
    rows0 = sid * ROWS_PT
    for tb in range(10):
        hgt = 64 if tb < 9 else ROWS_PT - 9 * 64
        pltpu.sync_copy(accS.at[pl.ds(rows0 + 64 * tb, hgt)],
                        acc_hbm.at[c, pl.ds(rows0 + 64 * tb, hgt)])
    plsc.subcore_barrier()


def _e_extras(sid, eap_hbm, pck_hbm, evs_hbm, ex_hbm, accS, ev_v, pck_v,
              row0, row1, eb0, eb1, srcb0, dstb0, srcb1, dstb1,
              e0, e1, s0, s1):
    _clear_accS(sid, row0, accS)
    _zero_rows(row1)

    lane0 = jnp.where(lax.iota(jnp.int32, 16) == 0, 1.0, 0.0)

    def eb_start(eb, sem, base, j):
        pltpu.async_copy(eap_hbm.at[pl.ds(base + j * (KB // 8), KB // 8)],
                         eb, sem)

    def eb_wait(eb, sem, base):
        pltpu.make_async_copy(eap_hbm.at[pl.ds(base, KB // 8)], eb, sem).wait()

    def build(row, eb, j):
        @pl.loop(0, KB)
        def _(k):
            w = plsc.load_gather(ev_v, [jnp.full((16,), j, jnp.int32),
                                        jnp.full((16,), k, jnp.int32)])
            ea16 = eb[lax.div(k, 8), pl.ds(DE * lax.rem(k, 8), DE)]
            row[k, pl.ds(0, 16)] = ea16 * w
            row[k, pl.ds(16, 16)] = w * lane0

    def s_start(row, dstb, sem):
        pltpu.async_copy(row, accS.at[dstb], sem, add=True)

    def s_wait(row, dstb, sem):
        pltpu.make_async_copy(row, accS.at[dstb], sem).wait()

    for rr in range(2):
        r = sid * 2 + rr
        base = r * (EPT_P // 8)
        pltpu.sync_copy(evs_hbm.at[r], ev_v)
        _load_pck_chunk(pck_hbm, r, pck_v, 0)
        _load_pck_chunk(pck_hbm, r, pck_v, 1)
        eb_start(eb0, e0, base, 0)

        @pl.loop(0, NB, step=2)
        def _(j):
            @pl.when(j > 0)
            def _():
                s_wait(row1, dstb1, s1)

            eb_start(eb1, e1, base, j + 1)
            eb_wait(eb0, e0, base)
            _unpack_idx(pck_v, j, srcb0, dstb0)
            build(row0, eb0, j)
            s_start(row0, dstb0, s0)
            eb_wait(eb1, e1, base)
            _unpack_idx(pck_v, j + 1, srcb1, dstb1)
            build(row1, eb1, j + 1)
            s_start(row1, dstb1, s1)

            @pl.when((lax.rem(j + 2, 8) == 0) & (j >= 14) & (j + 2 < NB))
            def _():
                _load_pck_chunk(pck_hbm, r, pck_v, lax.div(j + 2, 8))

            s_wait(row0, dstb0, s0)

            @pl.when(j + 2 < NB)
            def _():
                eb_start(eb0, e0, base, j + 2)

        s_wait(row1, dstb1, s1)

    plsc.subcore_barrier()
    rows0 = sid * ROWS_PT
    for tb in range(10):
        hgt = 64 if tb < 9 else ROWS_PT - 9 * 64
        pltpu.sync_copy(accS.at[pl.ds(rows0 + 64 * tb, hgt)],
                        ex_hbm.at[pl.ds(rows0 + 64 * tb, hgt)])
    plsc.subcore_barrier()


def _e_body(h8_hbm, pck_hbm, evs_hbm, eap_hbm, acc_hbm, ex_hbm,
            accS, pck_v, ev_v, row0, row1, eb0, eb1,
            srcb0, dstb0, srcb1, dstb1, g0, g1, s0, s1):
    cid = lax.axis_index("c")
    sid = lax.axis_index("s")

    for c in range(NCHUNK):
        @pl.when(cid == c // (NCHUNK // 2))
        def _():
            _e_round(c, sid, h8_hbm, pck_hbm, evs_hbm, acc_hbm, accS,
                     ev_v, pck_v,
                     row0, row1, srcb0, dstb0, srcb1, dstb1, g0, g1, s0, s1)

    @pl.when(cid == 0)
    def _():
        _e_extras(sid, eap_hbm, pck_hbm, evs_hbm, ex_hbm, accS, ev_v, pck_v,
                  row0, row1, eb0, eb1, srcb0, dstb0, srcb1, dstb1,
                  g0, g1, s0, s1)


def _kernel_e(h8, pck3, evs3, eap3):
    mesh = plsc.VectorSubcoreMesh(core_axis_name="c", subcore_axis_name="s")
    f = pl.kernel(
        _e_body,
        out_type=[
            jax.ShapeDtypeStruct((NCHUNK, NP, 128), jnp.float32),
            jax.ShapeDtypeStruct((NP, 128), jnp.float32),
        ],
        mesh=mesh,
        scratch_types=[
            pltpu.VMEM_SHARED((NP, 128), jnp.float32),
            pltpu.VMEM((2, 8, KB), jnp.int32),
            pltpu.VMEM((NB, KB), jnp.float32),
            pltpu.VMEM((KB, 128), jnp.float32),
            pltpu.VMEM((KB, 128), jnp.float32),
            pltpu.VMEM((KB // 8, 128), jnp.float32),
            pltpu.VMEM((KB // 8, 128), jnp.float32),
            pltpu.VMEM((KB,), jnp.int32),
            pltpu.VMEM((KB,), jnp.int32),
            pltpu.VMEM((KB,), jnp.int32),
            pltpu.VMEM((KB,), jnp.int32),
            pltpu.SemaphoreType.DMA,
            pltpu.SemaphoreType.DMA,
            pltpu.SemaphoreType.DMA,
            pltpu.SemaphoreType.DMA,
        ],
        compiler_params=_sc_compiler_params(),
    )
    return f(h8, pck3, evs3, eap3)


def _f1_body(acc_ref, ex_ref, we_ref, benc_ref,
             gg1_ref, bg1_ref, gg2_ref, bg2_ref,
             gn1_ref, bn1_ref, gn2_ref, bn2_ref,
             gate_ref, val_ref):
    ex = ex_ref[...]
    num = acc_ref[...] + jnp.dot(ex[:, :DE], we_ref[...],
                                 preferred_element_type=jnp.float32)
    den = ex[:, DE:DE + 1] + 1e-16
    xe = num / den + benc_ref[...]
    xe = jnp.where(xe > 0.0, xe, jnp.exp(jnp.minimum(xe, 0.0)) - 1.0)
    hg = jnp.maximum(jnp.dot(xe, gg1_ref[...], preferred_element_type=jnp.float32)
                     + bg1_ref[...], 0.0)
    gate_ref[...] = jnp.dot(hg, gg2_ref[...], preferred_element_type=jnp.float32) + bg2_ref[...]
    hv = jnp.maximum(jnp.dot(xe, gn1_ref[...], preferred_element_type=jnp.float32)
                     + bn1_ref[...], 0.0)
    val_ref[...] = jnp.dot(hv, gn2_ref[...], preferred_element_type=jnp.float32) + bn2_ref[...]


def _kernel_f1(acc, extras, We, b_enc, Gg1, bg1, Gg2, bg2, Gn1, bn1, Gn2, bn2):
    bn = 1000
    full = lambda a, b: pl.BlockSpec((a, b), lambda i: (0, 0))
    return pl.pallas_call(
        _f1_body,
        grid=(N // bn,),
        in_specs=[
            pl.BlockSpec((bn, EMB), lambda i: (i, 0)),
            pl.BlockSpec((bn, 32), lambda i: (i, 0)),
            full(DE, EMB), full(1, EMB),
            full(EMB, 512), full(1, 512), full(512, 1), full(1, 1),
            full(EMB, 512), full(1, 512), full(512, EMB), full(1, EMB),
        ],
        out_specs=[
            pl.BlockSpec((bn, 1), lambda i: (i, 0)),
            pl.BlockSpec((bn, EMB), lambda i: (i, 0)),
        ],
        out_shape=[
            jax.ShapeDtypeStruct((N, 1), jnp.float32),
            jax.ShapeDtypeStruct((N, EMB), jnp.float32),
        ],
    )(acc, extras, We, b_enc, Gg1, bg1, Gg2, bg2, Gn1, bn1, Gn2, bn2)


def _f2_body(gate_ref, batch_ref, val_ref, f1_ref, bf1_ref, f2_ref, bf2_ref,
             out_ref):
    g = gate_ref[...]
    m = jnp.max(g)
    ge = jnp.exp(g - m)
    gid = lax.broadcasted_iota(jnp.int32, (64, 1), 0)
    wmat = (batch_ref[...] == gid).astype(jnp.float32) * ge
    sseg = jnp.sum(wmat, axis=1, keepdims=True)
    pooled = jnp.dot(wmat, val_ref[...],
                     preferred_element_type=jnp.float32) / (sseg + 1e-16)
    hfc = jnp.maximum(pooled, 0.0)
    z = jax.nn.sigmoid(jnp.dot(hfc, f1_ref[...],
                               preferred_element_type=jnp.float32) + bf1_ref[...])
    out_ref[...] = jnp.dot(z, f2_ref[...],
                           preferred_element_type=jnp.float32) + bf2_ref[...]


def _kernel_f2(gate_row, batch_row, val, F1, bf1, F2, bf2):
    return pl.pallas_call(
        _f2_body,
        out_shape=jax.ShapeDtypeStruct((64, 3), jnp.float32),
    )(gate_row, batch_row, val, F1, bf1, F2, bf2)


def kernel(x, edge_index, edge_attr, batch, W, We, a_src, a_dst, a_edge, b_enc,
           Gg1, bg1, Gg2, bg2, Gn1, bn1, Gn2, bn2, F1, bf1, F2, bf2):
    src = edge_index[0]
    dst = edge_index[1]
    pad = EPT_P - EPT
    srcp = jnp.pad(src.reshape(NW, EPT), ((0, 0), (0, pad)))
    dstp = jnp.pad(dst.reshape(NW, EPT), ((0, 0), (0, pad)),
                   constant_values=DUMMY)
    pck = srcp | (dstp << 14)
    eap2 = jnp.pad(edge_attr.reshape(NW, EPT, DE),
                   ((0, 0), (0, pad), (0, 0))).reshape(NW * EPT_P * DE // 128,
                                                       128)
    a_sd = jnp.stack([a_src, a_dst], axis=1)

    h8, s_sd = _kernel_a(x, W, a_sd)
    lsd = _kernel_b(s_sd[:, 0], s_sd[:, 1], pck)
    t8 = _kernel_a2(eap2, We, a_edge.reshape(EMB, 1))
    ev = _kernel_c(lsd, t8.reshape(NW, EPT_P))
    acc3, extras = _kernel_e(h8,
                             pck.reshape(NW, NB, KB),
                             ev.reshape(NW, NB, KB),
                             eap2)
    acc = acc3.transpose(1, 0, 2).reshape(NP, EMB)
    gate, val = _kernel_f1(acc[:N], extras[:N, :32], We, b_enc.reshape(1, EMB),
                           Gg1, bg1.reshape(1, 512), Gg2, bg2.reshape(1, 1),
                           Gn1, bn1.reshape(1, 512), Gn2, bn2.reshape(1, EMB))
    out = _kernel_f2(gate.reshape(1, N), batch.reshape(1, N), val,
                     F1, bf1.reshape(1, 512), F2, bf2.reshape(1, 3))
    return out


# speedup vs baseline: 3.9853x; 1.0011x over previous
import dataclasses
import functools

import numpy as np

import jax
import jax.numpy as jnp
from jax import lax
from jax.experimental import pallas as pl
from jax.experimental.pallas import tpu as pltpu
from jax.experimental.pallas import tpu_sc as plsc

N = 10000
E = 320000
EMB = 1024
DE = 16
NW = 32
EPT = E // NW
KB = 64
NB = 160
EPT_P = NB * KB
NP = N + 112
DUMMY = N + 8
ROWS_PT = NP // 16
NCHUNK = 8

_PG = np.arange(32).reshape(2, 16).T.reshape(-1)
_PERM = (np.arange(0, EMB, 32)[:, None] + _PG[None, :]).reshape(-1)
_IPERM = np.argsort(_PERM)


def _sc_compiler_params():
    cp = pltpu.CompilerParams()
    if "needs_layout_passes" in pltpu.CompilerParams.__dataclass_fields__:
        cp = dataclasses.replace(cp, needs_layout_passes=False)
    return cp


def _a_body(x_ref, w_ref, asd_ref, h8_ref, ssd_ref):
    c = pl.program_id(1)
    hc = jnp.dot(x_ref[...], w_ref[0], preferred_element_type=jnp.float32)
    h8_ref[...] = hc[None]

    @pl.when(c == 0)
    def _():
        ssd_ref[...] = jnp.zeros_like(ssd_ref)

    ssd_ref[...] += jnp.dot(hc, asd_ref[...], preferred_element_type=jnp.float32)


def _kernel_a(x, W, a_sd):
    bn = 1000
    W3 = W.reshape(128, NCHUNK, 128).transpose(1, 0, 2)
    return pl.pallas_call(
        _a_body,
        grid=(N // bn, NCHUNK),
        in_specs=[
            pl.BlockSpec((bn, 128), lambda i, c: (i, 0)),
            pl.BlockSpec((1, 128, 128), lambda i, c: (c, 0, 0)),
            pl.BlockSpec((128, 2), lambda i, c: (c, 0)),
        ],
        out_specs=[
            pl.BlockSpec((1, bn, 128), lambda i, c: (c, i, 0)),
            pl.BlockSpec((bn, 2), lambda i, c: (i, 0)),
        ],
        out_shape=[
            jax.ShapeDtypeStruct((NCHUNK, N, 128), jnp.float32),
            jax.ShapeDtypeStruct((N, 2), jnp.float32),
        ],
    )(x, W3, a_sd)


def _b_body(ssrc_hbm, sdst_hbm, pck_hbm, lsd_hbm,
            ssrc_v, sdst_v, pck_v, l_v):
    wid = lax.axis_index("s") * 2 + lax.axis_index("c")
    pltpu.sync_copy(ssrc_hbm, ssrc_v)
    pltpu.sync_copy(sdst_hbm, sdst_v)
    pltpu.sync_copy(pck_hbm.at[wid], pck_v)

    @pl.loop(0, EPT_P, step=16)
    def _(i):
        p = pck_v[pl.ds(i, 16)]
        si = lax.bitwise_and(p, 0x3FFF)
        di = lax.bitwise_and(lax.shift_right_logical(p, 14), 0x3FFF)
        a = plsc.load_gather(ssrc_v, [si])
        b = plsc.load_gather(sdst_v, [di])
        l_v[pl.ds(i, 16)] = a + b

    pltpu.sync_copy(l_v, lsd_hbm.at[wid])


def _kernel_b(s_src, s_dst, pck):
    mesh = plsc.VectorSubcoreMesh(core_axis_name="c", subcore_axis_name="s")
    f = pl.kernel(
        _b_body,
        out_type=jax.ShapeDtypeStruct((NW, EPT_P), jnp.float32),
        mesh=mesh,
        scratch_types=[
            pltpu.VMEM((N,), jnp.float32),
            pltpu.VMEM((N,), jnp.float32),
            pltpu.VMEM((EPT_P,), jnp.int32),
            pltpu.VMEM((EPT_P,), jnp.float32),
        ],
        compiler_params=_sc_compiler_params(),
    )
    return f(s_src, s_dst, pck)


def _a2_body(eap_ref, we_ref, ae_ref, t8_ref):
    wea = jnp.dot(we_ref[...], ae_ref[...], preferred_element_type=jnp.float32)
    wea_t = jnp.broadcast_to(wea.reshape(1, DE), (8, DE)).reshape(128, 1)
    rowi = lax.broadcasted_iota(jnp.int32, (128, 8), 0)
    coli = lax.broadcasted_iota(jnp.int32, (128, 8), 1)
    wea8 = jnp.where(coli == rowi // DE, wea_t, 0.0)
    t8_ref[...] = jnp.dot(eap_ref[...], wea8, preferred_element_type=jnp.float32)


def _kernel_a2(eap2, We, a_edge):
    nr = NW * EPT_P // 8
    bn = nr // 10
    return pl.pallas_call(
        _a2_body,
        grid=(10,),
        in_specs=[
            pl.BlockSpec((bn, 128), lambda i: (i, 0)),
            pl.BlockSpec((DE, EMB), lambda i: (0, 0)),
            pl.BlockSpec((EMB, 1), lambda i: (0, 0)),
        ],
        out_specs=pl.BlockSpec((bn, 8), lambda i: (i, 0)),
        out_shape=jax.ShapeDtypeStruct((nr, 8), jnp.float32),
    )(eap2, We, a_edge)


def _c_body(lsd_ref, t_ref, ev_ref):
    l = lsd_ref[...] + t_ref[...]
    l = jnp.where(l >= 0.0, l, 0.2 * l)
    m = jnp.max(l)
    ev_ref[...] = jnp.exp(l - m)


def _kernel_c(lsd, t2d):
    return pl.pallas_call(
        _c_body,
        out_shape=jax.ShapeDtypeStruct((NW, EPT_P), jnp.float32),
    )(lsd, t2d)


def _zero_rows(row_v):
    nq = row_v.shape[1] // 16

    @pl.loop(0, row_v.shape[0])
    def _(rz):
        for q in range(nq):
            row_v[rz, pl.ds(16 * q, 16)] = jnp.zeros((16,), jnp.float32)


def _clear_accS(sid, row_v, accS):
    rows0 = sid * ROWS_PT
    _zero_rows(row_v)
    for tb in range(10):
        hgt = 64 if tb < 9 else ROWS_PT - 9 * 64
        pltpu.sync_copy(row_v.at[pl.ds(0, hgt)],
                        accS.at[pl.ds(rows0 + 64 * tb, hgt)])
    plsc.subcore_barrier()


def _unpack_idx(pck_v, j, srcb, dstb):
    slot = lax.rem(lax.div(j, 8), 2)
    jj = lax.rem(j, 8)
    for q in range(KB // 16):
        p = pck_v[slot, jj, pl.ds(16 * q, 16)]
        srcb[pl.ds(16 * q, 16)] = lax.bitwise_and(p, 0x3FFF)
        dstb[pl.ds(16 * q, 16)] = lax.bitwise_and(
            lax.shift_right_logical(p, 14), 0x3FFF)


def _load_pck_chunk(pck_hbm, r, pck_v, cidx):
    pltpu.sync_copy(pck_hbm.at[r, pl.ds(cidx * 8, 8)],
                    pck_v.at[lax.rem(cidx, 2)])


def _mult_rows(row, ev_v, j):
    @pl.loop(0, KB)
    def _(k):
        w = plsc.load_gather(ev_v, [jnp.full((16,), j, jnp.int32),
                                    jnp.full((16,), k, jnp.int32)])
        for q in range(8):
            row[k, pl.ds(16 * q, 16)] = row[k, pl.ds(16 * q, 16)] * w


def _e_round(c, sid, h8_hbm, pck_hbm, evs_hbm, acc_hbm, accS, ev_v, pck_v,
             row0, row1, srcb0, dstb0, srcb1, dstb1, g0, g1, s0, s1):
    _clear_accS(sid, row0, accS)
    hc = h8_hbm.at[c]

    def g_start(row, srcb, sem):
        for h in range(2):
            pltpu.async_copy(hc.at[srcb.at[pl.ds(32 * h, 32)]],
                             row.at[pl.ds(32 * h, 32)], sem)

    def g_wait(row, srcb, sem):
        for h in range(2):
            pltpu.make_async_copy(hc.at[srcb.at[pl.ds(32 * h, 32)]],
                                  row.at[pl.ds(32 * h, 32)], sem).wait()

    def s_start(row, dstb, sem):
        pltpu.async_copy(row, accS.at[dstb], sem, add=True)

    def s_wait(row, dstb, sem):
        pltpu.make_async_copy(row, accS.at[dstb], sem).wait()

    for rr in range(2):
        r = sid * 2 + rr
        pltpu.sync_copy(evs_hbm.at[r], ev_v)
        _load_pck_chunk(pck_hbm, r, pck_v, 0)
        _load_pck_chunk(pck_hbm, r, pck_v, 1)

        _unpack_idx(pck_v, 0, srcb0, dstb0)
        g_start(row0, srcb0, g0)

        @pl.loop(0, NB, step=2)
        def _(j):
            @pl.when(j > 0)
            def _():
                s_wait(row1, dstb1, s1)

            _unpack_idx(pck_v, j + 1, srcb1, dstb1)
            g_start(row1, srcb1, g1)
            g_wait(row0, srcb0, g0)
            _mult_rows(row0, ev_v, j)
            s_start(row0, dstb0, s0)
            g_wait(row1, srcb1, g1)
            _mult_rows(row1, ev_v, j + 1)
            s_start(row1, dstb1, s1)

            @pl.when((lax.rem(j + 2, 8) == 0) & (j >= 14) & (j + 2 < NB))
            def _():
                _load_pck_chunk(pck_hbm, r, pck_v, lax.div(j + 2, 8))

            s_wait(row0, dstb0, s0)

            @pl.when(j + 2 < NB)
            def _():
                _unpack_idx(pck_v, j + 2, srcb0, dstb0)
                g_start(row0, srcb0, g0)

        s_wait(row1, dstb1, s1)

    plsc.subcore_barrier()
    rows0 = sid * ROWS_PT
    for tb in range(10):
        hgt = 64 if tb < 9 else ROWS_PT - 9 * 64
        pltpu.sync_copy(accS.at[pl.ds(rows0 + 64 * tb, hgt)],
                        acc_hbm.at[c, pl.ds(rows0 + 64 * tb, hgt)])
    plsc.subcore_barrier()


def _e_extras(sid, eap_hbm, pck_hbm, evs_hbm, ex_hbm, accS, ev_v, pck_v,
              row0, row1, eb0, eb1, srcb0, dstb0, srcb1, dstb1,
              e0, e1, s0, s1):
    _clear_accS(sid, row0, accS)
    _zero_rows(row1)

    lane0 = jnp.where(lax.iota(jnp.int32, 16) == 0, 1.0, 0.0)

    def eb_start(eb, sem, base, j):
        pltpu.async_copy(eap_hbm.at[pl.ds(base + j * (KB // 8), KB // 8)],
                         eb, sem)

    def eb_wait(eb, sem, base):
        pltpu.make_async_copy(eap_hbm.at[pl.ds(base, KB // 8)], eb, sem).wait()

    def build(row, eb, j):
        @pl.loop(0, KB)
        def _(k):
            w = plsc.load_gather(ev_v, [jnp.full((16,), j, jnp.int32),
                                        jnp.full((16,), k, jnp.int32)])
            ea16 = eb[lax.div(k, 8), pl.ds(DE * lax.rem(k, 8), DE)]
            row[k, pl.ds(0, 16)] = ea16 * w
            row[k, pl.ds(16, 16)] = w * lane0

    def s_start(row, dstb, sem):
        pltpu.async_copy(row, accS.at[dstb], sem, add=True)

    def s_wait(row, dstb, sem):
        pltpu.make_async_copy(row, accS.at[dstb], sem).wait()

    for rr in range(2):
        r = sid * 2 + rr
        base = r * (EPT_P // 8)
        pltpu.sync_copy(evs_hbm.at[r], ev_v)
        _load_pck_chunk(pck_hbm, r, pck_v, 0)
        _load_pck_chunk(pck_hbm, r, pck_v, 1)
        eb_start(eb0, e0, base, 0)

        @pl.loop(0, NB, step=2)
        def _(j):
            @pl.when(j > 0)
            def _():
                s_wait(row1, dstb1, s1)

            eb_start(eb1, e1, base, j + 1)
            eb_wait(eb0, e0, base)
            _unpack_idx(pck_v, j, srcb0, dstb0)
            build(row0, eb0, j)
            s_start(row0, dstb0, s0)
            eb_wait(eb1, e1, base)
            _unpack_idx(pck_v, j + 1, srcb1, dstb1)
            build(row1, eb1, j + 1)
            s_start(row1, dstb1, s1)

            @pl.when((lax.rem(j + 2, 8) == 0) & (j >= 14) & (j + 2 < NB))
            def _():
                _load_pck_chunk(pck_hbm, r, pck_v, lax.div(j + 2, 8))

            s_wait(row0, dstb0, s0)

            @pl.when(j + 2 < NB)
            def _():
                eb_start(eb0, e0, base, j + 2)

        s_wait(row1, dstb1, s1)

    plsc.subcore_barrier()
    rows0 = sid * ROWS_PT
    for tb in range(10):
        hgt = 64 if tb < 9 else ROWS_PT - 9 * 64
        pltpu.sync_copy(accS.at[pl.ds(rows0 + 64 * tb, hgt)],
                        ex_hbm.at[pl.ds(rows0 + 64 * tb, hgt)])
    plsc.subcore_barrier()


def _e_body(h8_hbm, pck_hbm, evs_hbm, eap_hbm, acc_hbm, ex_hbm,
            accS, pck_v, ev_v, row0, row1, eb0, eb1,
            srcb0, dstb0, srcb1, dstb1, g0, g1, s0, s1):
    cid = lax.axis_index("c")
    sid = lax.axis_index("s")

    for c in range(NCHUNK):
        @pl.when(cid == c // (NCHUNK // 2))
        def _():
            _e_round(c, sid, h8_hbm, pck_hbm, evs_hbm, acc_hbm, accS,
                     ev_v, pck_v,
                     row0, row1, srcb0, dstb0, srcb1, dstb1, g0, g1, s0, s1)

    @pl.when(cid == 0)
    def _():
        _e_extras(sid, eap_hbm, pck_hbm, evs_hbm, ex_hbm, accS, ev_v, pck_v,
                  row0, row1, eb0, eb1, srcb0, dstb0, srcb1, dstb1,
                  g0, g1, s0, s1)


def _kernel_e(h8, pck3, evs3, eap3):
    mesh = plsc.VectorSubcoreMesh(core_axis_name="c", subcore_axis_name="s")
    f = pl.kernel(
        _e_body,
        out_type=[
            jax.ShapeDtypeStruct((NCHUNK, NP, 128), jnp.float32),
            jax.ShapeDtypeStruct((NP, 128), jnp.float32),
        ],
        mesh=mesh,
        scratch_types=[
            pltpu.VMEM_SHARED((NP, 128), jnp.float32),
            pltpu.VMEM((2, 8, KB), jnp.int32),
            pltpu.VMEM((NB, KB), jnp.float32),
            pltpu.VMEM((KB, 128), jnp.float32),
            pltpu.VMEM((KB, 128), jnp.float32),
            pltpu.VMEM((KB // 8, 128), jnp.float32),
            pltpu.VMEM((KB // 8, 128), jnp.float32),
            pltpu.VMEM((KB,), jnp.int32),
            pltpu.VMEM((KB,), jnp.int32),
            pltpu.VMEM((KB,), jnp.int32),
            pltpu.VMEM((KB,), jnp.int32),
            pltpu.SemaphoreType.DMA,
            pltpu.SemaphoreType.DMA,
            pltpu.SemaphoreType.DMA,
            pltpu.SemaphoreType.DMA,
        ],
        compiler_params=_sc_compiler_params(),
    )
    return f(h8, pck3, evs3, eap3)


def _f1_body(acc_ref, ex_ref, we_ref, benc_ref,
             gg1_ref, bg1_ref, gg2_ref, bg2_ref,
             gn1_ref, bn1_ref, gn2_ref, bn2_ref,
             gate_ref, val_ref):
    ex = ex_ref[...]
    num = acc_ref[...] + jnp.dot(ex[:, :DE], we_ref[...],
                                 preferred_element_type=jnp.float32)
    den = ex[:, DE:DE + 1] + 1e-16
    xe = num / den + benc_ref[...]
    xe = jnp.where(xe > 0.0, xe, jnp.exp(jnp.minimum(xe, 0.0)) - 1.0)
    hg = jnp.maximum(jnp.dot(xe, gg1_ref[...], preferred_element_type=jnp.float32)
                     + bg1_ref[...], 0.0)
    gate_ref[...] = jnp.dot(hg, gg2_ref[...], preferred_element_type=jnp.float32) + bg2_ref[...]
    hv = jnp.maximum(jnp.dot(xe, gn1_ref[...], preferred_element_type=jnp.float32)
                     + bn1_ref[...], 0.0)
    val_ref[...] = jnp.dot(hv, gn2_ref[...], preferred_element_type=jnp.float32) + bn2_ref[...]


def _kernel_f1(acc, extras, We, b_enc, Gg1, bg1, Gg2, bg2, Gn1, bn1, Gn2, bn2):
    bn = 1000
    full = lambda a, b: pl.BlockSpec((a, b), lambda i: (0, 0))
    return pl.pallas_call(
        _f1_body,
        grid=(N // bn,),
        in_specs=[
            pl.BlockSpec((bn, EMB), lambda i: (i, 0)),
            pl.BlockSpec((bn, 32), lambda i: (i, 0)),
            full(DE, EMB), full(1, EMB),
            full(EMB, 512), full(1, 512), full(512, 1), full(1, 1),
            full(EMB, 512), full(1, 512), full(512, EMB), full(1, EMB),
        ],
        out_specs=[
            pl.BlockSpec((bn, 1), lambda i: (i, 0)),
            pl.BlockSpec((bn, EMB), lambda i: (i, 0)),
        ],
        out_shape=[
            jax.ShapeDtypeStruct((N, 1), jnp.float32),
            jax.ShapeDtypeStruct((N, EMB), jnp.float32),
        ],
    )(acc, extras, We, b_enc, Gg1, bg1, Gg2, bg2, Gn1, bn1, Gn2, bn2)


def _f2_body(gate_ref, batch_ref, val_ref, f1_ref, bf1_ref, f2_ref, bf2_ref,
             out_ref):
    g = gate_ref[...]
    m = jnp.max(g)
    ge = jnp.exp(g - m)
    gid = lax.broadcasted_iota(jnp.int32, (64, 1), 0)
    wmat = (batch_ref[...] == gid).astype(jnp.float32) * ge
    sseg = jnp.sum(wmat, axis=1, keepdims=True)
    pooled = jnp.dot(wmat, val_ref[...],
                     preferred_element_type=jnp.float32) / (sseg + 1e-16)
    hfc = jnp.maximum(pooled, 0.0)
    z = jax.nn.sigmoid(jnp.dot(hfc, f1_ref[...],
                               preferred_element_type=jnp.float32) + bf1_ref[...])
    out_ref[...] = jnp.dot(z, f2_ref[...],
                           preferred_element_type=jnp.float32) + bf2_ref[...]


def _kernel_f2(gate_row, batch_row, val, F1, bf1, F2, bf2):
    return pl.pallas_call(
        _f2_body,
        out_shape=jax.ShapeDtypeStruct((64, 3), jnp.float32),
    )(gate_row, batch_row, val, F1, bf1, F2, bf2)


def kernel(x, edge_index, edge_attr, batch, W, We, a_src, a_dst, a_edge, b_enc,
           Gg1, bg1, Gg2, bg2, Gn1, bn1, Gn2, bn2, F1, bf1, F2, bf2):
    src = edge_index[0]
    dst = edge_index[1]
    pad = EPT_P - EPT
    srcp = jnp.pad(src.reshape(NW, EPT), ((0, 0), (0, pad)))
    dstp = jnp.pad(dst.reshape(NW, EPT), ((0, 0), (0, pad)),
                   constant_values=DUMMY)
    pck = srcp | (dstp << 14)
    eap2 = jnp.pad(edge_attr.reshape(NW, EPT, DE),
                   ((0, 0), (0, pad), (0, 0))).reshape(NW * EPT_P * DE // 128,
                                                       128)
    a_sd = jnp.stack([a_src, a_dst], axis=1)

    h8, s_sd = _kernel_a(x, W, a_sd)
    lsd = _kernel_b(s_sd[:, 0], s_sd[:, 1], pck)
    t8 = _kernel_a2(eap2, We, a_edge.reshape(EMB, 1))
    ev = _kernel_c(lsd, t8.reshape(NW, EPT_P))
    acc3, extras = _kernel_e(h8,
                             pck.reshape(NW, NB, KB),
                             ev.reshape(NW, NB, KB),
                             eap2)
    acc = acc3.transpose(1, 0, 2).reshape(NP, EMB)
    gate, val = _kernel_f1(acc[:N], extras[:N, :32], We, b_enc.reshape(1, EMB),
                           Gg1, bg1.reshape(1, 512), Gg2, bg2.reshape(1, 1),
                           Gn1, bn1.reshape(1, 512), Gn2, bn2.reshape(1, EMB))
    out = _kernel_f2(gate.reshape(1, N), batch.reshape(1, N), val,
                     F1, bf1.reshape(1, 512), F2, bf2.reshape(1, 3))
    return out


# speedup vs baseline: 3.9878x; 1.0006x over previous
import dataclasses

import jax
import jax.numpy as jnp
from jax import lax
from jax.experimental import pallas as pl
from jax.experimental.pallas import tpu as pltpu
from jax.experimental.pallas import tpu_sc as plsc

N = 10000
E = 320000
EMB = 1024
DE = 16
NW = 32
EPT = E // NW
KB = 64
NB = 160
EPT_P = NB * KB
NP = N + 112
DUMMY = N + 8
ROWS_PT = NP // 16
NCHUNK = 8


def _sc_compiler_params():
    cp = pltpu.CompilerParams()
    if "needs_layout_passes" in pltpu.CompilerParams.__dataclass_fields__:
        cp = dataclasses.replace(cp, needs_layout_passes=False)
    return cp


def _a_body(x_ref, w_ref, asd_ref, h8_ref, ssd_ref):
    c = pl.program_id(1)
    hc = jnp.dot(x_ref[...], w_ref[0], preferred_element_type=jnp.float32)
    h8_ref[...] = hc[None]

    @pl.when(c == 0)
    def _():
        ssd_ref[...] = jnp.zeros_like(ssd_ref)

    ssd_ref[...] += jnp.dot(hc, asd_ref[...], preferred_element_type=jnp.float32)


def _kernel_a(x, W, a_sd):
    bn = 1000
    W3 = W.reshape(128, NCHUNK, 128).transpose(1, 0, 2)
    return pl.pallas_call(
        _a_body,
        grid=(N // bn, NCHUNK),
        in_specs=[
            pl.BlockSpec((bn, 128), lambda i, c: (i, 0)),
            pl.BlockSpec((1, 128, 128), lambda i, c: (c, 0, 0)),
            pl.BlockSpec((128, 2), lambda i, c: (c, 0)),
        ],
        out_specs=[
            pl.BlockSpec((1, bn, 128), lambda i, c: (c, i, 0)),
            pl.BlockSpec((bn, 2), lambda i, c: (i, 0)),
        ],
        out_shape=[
            jax.ShapeDtypeStruct((NCHUNK, N, 128), jnp.float32),
            jax.ShapeDtypeStruct((N, 2), jnp.float32),
        ],
    )(x, W3, a_sd)


def _b_body(ssrc_hbm, sdst_hbm, pck_hbm, lsd_hbm,
            ssrc_v, sdst_v, pck_v, l_v):
    wid = lax.axis_index("s") * 2 + lax.axis_index("c")
    pltpu.sync_copy(ssrc_hbm, ssrc_v)
    pltpu.sync_copy(sdst_hbm, sdst_v)
    pltpu.sync_copy(pck_hbm.at[wid], pck_v)

    @pl.loop(0, EPT_P, step=16)
    def _(i):
        p = pck_v[pl.ds(i, 16)]
        si = lax.bitwise_and(p, 0x3FFF)
        di = lax.bitwise_and(lax.shift_right_logical(p, 14), 0x3FFF)
        a = plsc.load_gather(ssrc_v, [si])
        b = plsc.load_gather(sdst_v, [di])
        l_v[pl.ds(i, 16)] = a + b

    pltpu.sync_copy(l_v, lsd_hbm.at[wid])


def _kernel_b(s_src, s_dst, pck):
    mesh = plsc.VectorSubcoreMesh(core_axis_name="c", subcore_axis_name="s")
    f = pl.kernel(
        _b_body,
        out_type=jax.ShapeDtypeStruct((NW, EPT_P), jnp.float32),
        mesh=mesh,
        scratch_types=[
            pltpu.VMEM((N,), jnp.float32),
            pltpu.VMEM((N,), jnp.float32),
            pltpu.VMEM((EPT_P,), jnp.int32),
            pltpu.VMEM((EPT_P,), jnp.float32),
        ],
        compiler_params=_sc_compiler_params(),
    )
    return f(s_src, s_dst, pck)


def _a2_body(eap_ref, we_ref, ae_ref, t8_ref):
    wea = jnp.dot(we_ref[...], ae_ref[...], preferred_element_type=jnp.float32)
    wea_t = jnp.broadcast_to(wea.reshape(1, DE), (8, DE)).reshape(128, 1)
    rowi = lax.broadcasted_iota(jnp.int32, (128, 8), 0)
    coli = lax.broadcasted_iota(jnp.int32, (128, 8), 1)
    wea8 = jnp.where(coli == rowi // DE, wea_t, 0.0)
    t8_ref[...] = jnp.dot(eap_ref[...], wea8, preferred_element_type=jnp.float32)


def _kernel_a2(eap2, We, a_edge):
    nr = NW * EPT_P // 8
    bn = nr // 10
    return pl.pallas_call(
        _a2_body,
        grid=(10,),
        in_specs=[
            pl.BlockSpec((bn, 128), lambda i: (i, 0)),
            pl.BlockSpec((DE, EMB), lambda i: (0, 0)),
            pl.BlockSpec((EMB, 1), lambda i: (0, 0)),
        ],
        out_specs=pl.BlockSpec((bn, 8), lambda i: (i, 0)),
        out_shape=jax.ShapeDtypeStruct((nr, 8), jnp.float32),
    )(eap2, We, a_edge)


def _c_body(lsd_ref, t_ref, ev_ref):
    l = lsd_ref[...] + t_ref[...]
    l = jnp.where(l >= 0.0, l, 0.2 * l)
    m = jnp.max(l)
    ev_ref[...] = jnp.exp(l - m)


def _kernel_c(lsd, t2d):
    return pl.pallas_call(
        _c_body,
        out_shape=jax.ShapeDtypeStruct((NW, EPT_P), jnp.float32),
    )(lsd, t2d)


def _zero_rows(row_v):
    nq = row_v.shape[1] // 16

    @pl.loop(0, row_v.shape[0])
    def _(rz):
        for q in range(nq):
            row_v[rz, pl.ds(16 * q, 16)] = jnp.zeros((16,), jnp.float32)


def _clear_accS(sid, row_v, accS):
    rows0 = sid * ROWS_PT
    _zero_rows(row_v)
    for tb in range(10):
        hgt = 64 if tb < 9 else ROWS_PT - 9 * 64
        pltpu.sync_copy(row_v.at[pl.ds(0, hgt)],
                        accS.at[pl.ds(rows0 + 64 * tb, hgt)])
    plsc.subcore_barrier()


def _unpack_idx(pck_v, j, srcb, dstb):
    slot = lax.rem(lax.div(j, 8), 2)
    jj = lax.rem(j, 8)
    for q in range(KB // 16):
        p = pck_v[slot, jj, pl.ds(16 * q, 16)]
        srcb[pl.ds(16 * q, 16)] = lax.bitwise_and(p, 0x3FFF)
        dstb[pl.ds(16 * q, 16)] = lax.bitwise_and(
            lax.shift_right_logical(p, 14), 0x3FFF)


def _load_pck_chunk(pck_hbm, r, pck_v, cidx):
    pltpu.sync_copy(pck_hbm.at[r, pl.ds(cidx * 8, 8)],
                    pck_v.at[lax.rem(cidx, 2)])


def _mult_rows(row, ev_v, j):
    @pl.loop(0, KB)
    def _(k):
        w = plsc.load_gather(ev_v, [jnp.full((16,), j, jnp.int32),
                                    jnp.full((16,), k, jnp.int32)])
        for q in range(8):
            row[k, pl.ds(16 * q, 16)] = row[k, pl.ds(16 * q, 16)] * w


def _e_round(c, sid, h8_hbm, pck_hbm, evs_hbm, acc_hbm, accS, ev_v, pck_v,
             row0, row1, srcb0, dstb0, srcb1, dstb1, g0, g1, s0, s1):
    _clear_accS(sid, row0, accS)
    hc = h8_hbm.at[c]

    def g_start(row, srcb, sem):
        for h in range(2):
            pltpu.async_copy(hc.at[srcb.at[pl.ds(32 * h, 32)]],
                             row.at[pl.ds(32 * h, 32)], sem)

    def g_wait(row, srcb, sem):
        for h in range(2):
            pltpu.make_async_copy(hc.at[srcb.at[pl.ds(32 * h, 32)]],
                                  row.at[pl.ds(32 * h, 32)], sem).wait()

    def s_start(row, dstb, sem):
        pltpu.async_copy(row, accS.at[dstb], sem, add=True)

    def s_wait(row, dstb, sem):
        pltpu.make_async_copy(row, accS.at[dstb], sem).wait()

    for rr in range(2):
        r = sid * 2 + rr
        pltpu.sync_copy(evs_hbm.at[r], ev_v)
        _load_pck_chunk(pck_hbm, r, pck_v, 0)
        _load_pck_chunk(pck_hbm, r, pck_v, 1)

        _unpack_idx(pck_v, 0, srcb0, dstb0)
        g_start(row0, srcb0, g0)

        @pl.loop(0, NB, step=2)
        def _(j):
            @pl.when(j > 0)
            def _():
                s_wait(row1, dstb1, s1)

            _unpack_idx(pck_v, j + 1, srcb1, dstb1)
            g_start(row1, srcb1, g1)
            g_wait(row0, srcb0, g0)
            _mult_rows(row0, ev_v, j)
            s_start(row0, dstb0, s0)
            g_wait(row1, srcb1, g1)
            _mult_rows(row1, ev_v, j + 1)
            s_start(row1, dstb1, s1)

            @pl.when((lax.rem(j + 2, 8) == 0) & (j >= 14) & (j + 2 < NB))
            def _():
                _load_pck_chunk(pck_hbm, r, pck_v, lax.div(j + 2, 8))

            s_wait(row0, dstb0, s0)

            @pl.when(j + 2 < NB)
            def _():
                _unpack_idx(pck_v, j + 2, srcb0, dstb0)
                g_start(row0, srcb0, g0)

        s_wait(row1, dstb1, s1)

    plsc.subcore_barrier()
    rows0 = sid * ROWS_PT
    for tb in range(10):
        hgt = 64 if tb < 9 else ROWS_PT - 9 * 64
        pltpu.sync_copy(accS.at[pl.ds(rows0 + 64 * tb, hgt)],
                        acc_hbm.at[c, pl.ds(rows0 + 64 * tb, hgt)])
    plsc.subcore_barrier()


def _e_extras(sid, eap_hbm, pck_hbm, evs_hbm, ex_hbm, accS, ev_v, pck_v,
              row0, row1, eb0, eb1, srcb0, dstb0, srcb1, dstb1,
              e0, e1, s0, s1):
    _clear_accS(sid, row0, accS)
    _zero_rows(row1)

    lane0 = jnp.where(lax.iota(jnp.int32, 16) == 0, 1.0, 0.0)

    def eb_start(eb, sem, base, j):
        pltpu.async_copy(eap_hbm.at[pl.ds(base + j * (KB // 8), KB // 8)],
                         eb, sem)

    def eb_wait(eb, sem, base):
        pltpu.make_async_copy(eap_hbm.at[pl.ds(base, KB // 8)], eb, sem).wait()

    def build(row, eb, j):
        @pl.loop(0, KB)
        def _(k):
            w = plsc.load_gather(ev_v, [jnp.full((16,), j, jnp.int32),
                                        jnp.full((16,), k, jnp.int32)])
            ea16 = eb[lax.div(k, 8), pl.ds(DE * lax.rem(k, 8), DE)]
            row[k, pl.ds(0, 16)] = ea16 * w
            row[k, pl.ds(16, 16)] = w * lane0

    def s_start(row, dstb, sem):
        pltpu.async_copy(row, accS.at[dstb], sem, add=True)

    def s_wait(row, dstb, sem):
        pltpu.make_async_copy(row, accS.at[dstb], sem).wait()

    for rr in range(2):
        r = sid * 2 + rr
        base = r * (EPT_P // 8)
        pltpu.sync_copy(evs_hbm.at[r], ev_v)
        _load_pck_chunk(pck_hbm, r, pck_v, 0)
        _load_pck_chunk(pck_hbm, r, pck_v, 1)
        eb_start(eb0, e0, base, 0)

        @pl.loop(0, NB, step=2)
        def _(j):
            @pl.when(j > 0)
            def _():
                s_wait(row1, dstb1, s1)

            eb_start(eb1, e1, base, j + 1)
            eb_wait(eb0, e0, base)
            _unpack_idx(pck_v, j, srcb0, dstb0)
            build(row0, eb0, j)
            s_start(row0, dstb0, s0)
            eb_wait(eb1, e1, base)
            _unpack_idx(pck_v, j + 1, srcb1, dstb1)
            build(row1, eb1, j + 1)
            s_start(row1, dstb1, s1)

            @pl.when((lax.rem(j + 2, 8) == 0) & (j >= 14) & (j + 2 < NB))
            def _():
                _load_pck_chunk(pck_hbm, r, pck_v, lax.div(j + 2, 8))

            s_wait(row0, dstb0, s0)

            @pl.when(j + 2 < NB)
            def _():
                eb_start(eb0, e0, base, j + 2)

        s_wait(row1, dstb1, s1)

    plsc.subcore_barrier()
    rows0 = sid * ROWS_PT
    for tb in range(10):
        hgt = 64 if tb < 9 else ROWS_PT - 9 * 64
        pltpu.sync_copy(accS.at[pl.ds(rows0 + 64 * tb, hgt)],
                        ex_hbm.at[pl.ds(rows0 + 64 * tb, hgt)])
    plsc.subcore_barrier()


def _e_body(h8_hbm, pck_hbm, evs_hbm, eap_hbm, acc_hbm, ex_hbm,
            accS, pck_v, ev_v, row0, row1, eb0, eb1,
            srcb0, dstb0, srcb1, dstb1, g0, g1, s0, s1):
    cid = lax.axis_index("c")
    sid = lax.axis_index("s")

    for c in range(NCHUNK):
        @pl.when(cid == c // (NCHUNK // 2))
        def _():
            _e_round(c, sid, h8_hbm, pck_hbm, evs_hbm, acc_hbm, accS,
                     ev_v, pck_v,
                     row0, row1, srcb0, dstb0, srcb1, dstb1, g0, g1, s0, s1)

    @pl.when(cid == 0)
    def _():
        _e_extras(sid, eap_hbm, pck_hbm, evs_hbm, ex_hbm, accS, ev_v, pck_v,
                  row0, row1, eb0, eb1, srcb0, dstb0, srcb1, dstb1,
                  g0, g1, s0, s1)


def _kernel_e(h8, pck3, evs3, eap3):
    mesh = plsc.VectorSubcoreMesh(core_axis_name="c", subcore_axis_name="s")
    f = pl.kernel(
        _e_body,
        out_type=[
            jax.ShapeDtypeStruct((NCHUNK, NP, 128), jnp.float32),
            jax.ShapeDtypeStruct((NP, 128), jnp.float32),
        ],
        mesh=mesh,
        scratch_types=[
            pltpu.VMEM_SHARED((NP, 128), jnp.float32),
            pltpu.VMEM((2, 8, KB), jnp.int32),
            pltpu.VMEM((NB, KB), jnp.float32),
            pltpu.VMEM((KB, 128), jnp.float32),
            pltpu.VMEM((KB, 128), jnp.float32),
            pltpu.VMEM((KB // 8, 128), jnp.float32),
            pltpu.VMEM((KB // 8, 128), jnp.float32),
            pltpu.VMEM((KB,), jnp.int32),
            pltpu.VMEM((KB,), jnp.int32),
            pltpu.VMEM((KB,), jnp.int32),
            pltpu.VMEM((KB,), jnp.int32),
            pltpu.SemaphoreType.DMA,
            pltpu.SemaphoreType.DMA,
            pltpu.SemaphoreType.DMA,
            pltpu.SemaphoreType.DMA,
        ],
        compiler_params=_sc_compiler_params(),
    )
    return f(h8, pck3, evs3, eap3)


def _f1_body(acc_ref, ex_ref, we_ref, benc_ref,
             gg1_ref, bg1_ref, gg2_ref, bg2_ref,
             gn1_ref, bn1_ref, gn2_ref, bn2_ref,
             gate_ref, val_ref):
    ex = ex_ref[...]
    num = acc_ref[...] + jnp.dot(ex[:, :DE], we_ref[...],
                                 preferred_element_type=jnp.float32)
    den = ex[:, DE:DE + 1] + 1e-16
    xe = num / den + benc_ref[...]
    xe = jnp.where(xe > 0.0, xe, jnp.exp(jnp.minimum(xe, 0.0)) - 1.0)
    hg = jnp.maximum(jnp.dot(xe, gg1_ref[...], preferred_element_type=jnp.float32)
                     + bg1_ref[...], 0.0)
    gate_ref[...] = jnp.dot(hg, gg2_ref[...], preferred_element_type=jnp.float32) + bg2_ref[...]
    hv = jnp.maximum(jnp.dot(xe, gn1_ref[...], preferred_element_type=jnp.float32)
                     + bn1_ref[...], 0.0)
    val_ref[...] = jnp.dot(hv, gn2_ref[...], preferred_element_type=jnp.float32) + bn2_ref[...]


def _kernel_f1(acc, extras, We, b_enc, Gg1, bg1, Gg2, bg2, Gn1, bn1, Gn2, bn2):
    bn = 1000
    full = lambda a, b: pl.BlockSpec((a, b), lambda i: (0, 0))
    return pl.pallas_call(
        _f1_body,
        grid=(N // bn,),
        in_specs=[
            pl.BlockSpec((bn, EMB), lambda i: (i, 0)),
            pl.BlockSpec((bn, 32), lambda i: (i, 0)),
            full(DE, EMB), full(1, EMB),
            full(EMB, 512), full(1, 512), full(512, 1), full(1, 1),
            full(EMB, 512), full(1, 512), full(512, EMB), full(1, EMB),
        ],
        out_specs=[
            pl.BlockSpec((bn, 1), lambda i: (i, 0)),
            pl.BlockSpec((bn, EMB), lambda i: (i, 0)),
        ],
        out_shape=[
            jax.ShapeDtypeStruct((N, 1), jnp.float32),
            jax.ShapeDtypeStruct((N, EMB), jnp.float32),
        ],
    )(acc, extras, We, b_enc, Gg1, bg1, Gg2, bg2, Gn1, bn1, Gn2, bn2)


def _f2_body(gate_ref, batch_ref, val_ref, f1_ref, bf1_ref, f2_ref, bf2_ref,
             out_ref):
    g = gate_ref[...]
    m = jnp.max(g)
    ge = jnp.exp(g - m)
    gid = lax.broadcasted_iota(jnp.int32, (64, 1), 0)
    wmat = (batch_ref[...] == gid).astype(jnp.float32) * ge
    sseg = jnp.sum(wmat, axis=1, keepdims=True)
    pooled = jnp.dot(wmat, val_ref[...],
                     preferred_element_type=jnp.float32) / (sseg + 1e-16)
    hfc = jnp.maximum(pooled, 0.0)
    z = jax.nn.sigmoid(jnp.dot(hfc, f1_ref[...],
                               preferred_element_type=jnp.float32) + bf1_ref[...])
    out_ref[...] = jnp.dot(z, f2_ref[...],
                           preferred_element_type=jnp.float32) + bf2_ref[...]


def _kernel_f2(gate_row, batch_row, val, F1, bf1, F2, bf2):
    return pl.pallas_call(
        _f2_body,
        out_shape=jax.ShapeDtypeStruct((64, 3), jnp.float32),
    )(gate_row, batch_row, val, F1, bf1, F2, bf2)


def kernel(x, edge_index, edge_attr, batch, W, We, a_src, a_dst, a_edge, b_enc,
           Gg1, bg1, Gg2, bg2, Gn1, bn1, Gn2, bn2, F1, bf1, F2, bf2):
    src = edge_index[0]
    dst = edge_index[1]
    pad = EPT_P - EPT
    srcp = jnp.pad(src.reshape(NW, EPT), ((0, 0), (0, pad)))
    dstp = jnp.pad(dst.reshape(NW, EPT), ((0, 0), (0, pad)),
                   constant_values=DUMMY)
    pck = srcp | (dstp << 14)
    eap2 = jnp.pad(edge_attr.reshape(NW, EPT, DE),
                   ((0, 0), (0, pad), (0, 0))).reshape(NW * EPT_P * DE // 128,
                                                       128)
    a_sd = jnp.stack([a_src, a_dst], axis=1)

    h8, s_sd = _kernel_a(x, W, a_sd)
    lsd = _kernel_b(s_sd[:, 0], s_sd[:, 1], pck)
    t8 = _kernel_a2(eap2, We, a_edge.reshape(EMB, 1))
    ev = _kernel_c(lsd, t8.reshape(NW, EPT_P))
    acc3, extras = _kernel_e(h8,
                             pck.reshape(NW, NB, KB),
                             ev.reshape(NW, NB, KB),
                             eap2)
    acc = acc3.transpose(1, 0, 2).reshape(NP, EMB)
    gate, val = _kernel_f1(acc[:N], extras[:N, :32], We, b_enc.reshape(1, EMB),
                           Gg1, bg1.reshape(1, 512), Gg2, bg2.reshape(1, 1),
                           Gn1, bn1.reshape(1, 512), Gn2, bn2.reshape(1, EMB))
    out = _kernel_f2(gate.reshape(1, N), batch.reshape(1, N), val,
                     F1, bf1.reshape(1, 512), F2, bf2.reshape(1, 3))
    return out
